# baseline (speedup 1.0000x reference)
import functools

import jax
import jax.numpy as jnp
from jax import lax
from jax.experimental import pallas as pl
from jax.experimental.pallas import tpu as pltpu
from jax.experimental.pallas import tpu_sc as plsc

EMB = 16
HID = 80
ND = 8
NC = 2
NS = 16
NW = NC * NS
CHUNK = 128


def _sc_gather_fn(B):
    bpw = B // NW
    nch = bpw // CHUNK
    mesh = plsc.VectorSubcoreMesh(
        core_axis_name="c", subcore_axis_name="s", num_cores=NC, num_subcores=NS
    )

    @functools.partial(
        pl.kernel,
        mesh=mesh,
        out_type=[jax.ShapeDtypeStruct((B, EMB), jnp.float32) for _ in range(5)],
        scratch_types=[pltpu.VMEM((5, nch, CHUNK), jnp.int32)]
        + [pltpu.VMEM((bpw, EMB), jnp.float32) for _ in range(5)]
        + [pltpu.SemaphoreType.DMA],
        compiler_params=pltpu.CompilerParams(use_tc_tiling_on_sc=False),
    )
    def body(idx_hbm, t0, t1, t2, t3, t4, o0, o1, o2, o3, o4, idxv,
             r0, r1, r2, r3, r4, sem):
        wid = lax.axis_index("s") * NC + lax.axis_index("c")
        base = wid * bpw
        pltpu.sync_copy(idx_hbm.at[wid], idxv)
        copies = []
        for f, (tab, rows) in enumerate(
            zip((t0, t1, t2, t3, t4), (r0, r1, r2, r3, r4))
        ):
            for j in range(nch):
                copies.append(
                    pltpu.async_copy(
                        tab.at[idxv.at[f, j]],
                        rows.at[pl.ds(j * CHUNK, CHUNK)],
                        sem,
                    )
                )
        for c in copies:
            c.wait()
        for rows, out in zip((r0, r1, r2, r3, r4), (o0, o1, o2, o3, o4)):
            pltpu.sync_copy(rows, out.at[pl.ds(base, bpw)])

    return body


def _ffn_body(e0, e1, e2, e3, e4, pidr, dk1t, sk1t, db1r, sb1t, dk2r, sk2t,
              db2r, sb2r, dlw, dlb, out):
    blk = out.shape[0]
    e = jnp.concatenate(
        [jnp.maximum(r[...], 0.0) for r in (e0, e1, e2, e3, e4)], axis=1
    )
    d = pidr[...] - 1
    k1 = sk1t[...] * dk1t[...]
    b1 = sb1t[...] + db1r[...]
    h = jnp.maximum(jnp.dot(e, k1, preferred_element_type=jnp.float32) + b1, 0.0)
    col = lax.broadcasted_iota(jnp.int32, (blk, ND * HID), 1) // HID
    hm = jnp.where(col == d, h, 0.0)
    k2 = sk2t[...] * dk2r[...]
    logits = jnp.dot(hm, k2, preferred_element_type=jnp.float32)
    oh = (lax.broadcasted_iota(jnp.int32, (blk, ND), 1) == d).astype(jnp.float32)
    b2 = sb2r[...] + db2r[...]
    logits = logits + jnp.dot(oh, b2, preferred_element_type=jnp.float32)
    ep = jnp.maximum(e0[...], 0.0)
    dl = jnp.sum(ep * dlw[...], axis=1, keepdims=True) + dlb[...]
    out[...] = logits + dl


def kernel(pid, uid, iid, cid, bid, batch_size, emb_pid, emb_uid, emb_iid,
           emb_cid, emb_bid, dk1, db1, dk2, db2, sk1, sb1, sk2, sb2, dl_w,
           dl_b):
    B = pid.shape[0]
    bpw = B // NW
    nch = bpw // CHUNK

    idx_all = (
        jnp.stack([pid, uid, iid, cid, bid])
        .astype(jnp.int32)
        .reshape(5, NW, nch, CHUNK)
        .transpose(1, 0, 2, 3)
    )

    e0, e1, e2, e3, e4 = _sc_gather_fn(B)(
        idx_all, emb_pid, emb_uid, emb_iid, emb_cid, emb_bid
    )

    dk1t = dk1.transpose(1, 0, 2).reshape(HID, ND * HID)
    sk1t = jnp.tile(sk1, (1, ND))
    db1r = db1.reshape(1, ND * HID)
    sb1t = jnp.tile(sb1.reshape(1, HID), (1, ND))
    dk2r = dk2.reshape(ND * HID, HID)
    sk2t = jnp.tile(sk2, (ND, 1))
    db2r = db2.reshape(ND, HID)
    sb2r = sb2.reshape(1, HID)
    dlw = dl_w.reshape(1, EMB)
    dlb = dl_b.reshape(1, 1)
    pid2 = pid.astype(jnp.int32).reshape(B, 1)

    BLK = 512
    grid = B // BLK
    full = lambda b: (0, 0)
    row = lambda b: (b, 0)
    return pl.pallas_call(
        _ffn_body,
        grid=(grid,),
        in_specs=[
            pl.BlockSpec((BLK, EMB), row),
            pl.BlockSpec((BLK, EMB), row),
            pl.BlockSpec((BLK, EMB), row),
            pl.BlockSpec((BLK, EMB), row),
            pl.BlockSpec((BLK, EMB), row),
            pl.BlockSpec((BLK, 1), row),
            pl.BlockSpec((HID, ND * HID), full),
            pl.BlockSpec((HID, ND * HID), full),
            pl.BlockSpec((1, ND * HID), full),
            pl.BlockSpec((1, ND * HID), full),
            pl.BlockSpec((ND * HID, HID), full),
            pl.BlockSpec((ND * HID, HID), full),
            pl.BlockSpec((ND, HID), full),
            pl.BlockSpec((1, HID), full),
            pl.BlockSpec((1, EMB), full),
            pl.BlockSpec((1, 1), full),
        ],
        out_specs=pl.BlockSpec((BLK, HID), row),
        out_shape=jax.ShapeDtypeStruct((B, HID), jnp.float32),
        compiler_params=pltpu.CompilerParams(
            dimension_semantics=("parallel",)
        ),
    )(e0, e1, e2, e3, e4, pid2, dk1t, sk1t, db1r, sb1t, dk2r, sk2t, db2r,
      sb2r, dlw, dlb)
